# baseline (speedup 1.0000x reference)
import jax
import jax.numpy as jnp
from jax.experimental import pallas as pl

_B, _C, _F, _N = 4, 128, 12, 512


def _body(y_ref, x_ref, we_ref, wl_ref, b_ref, o_ref):
    b = pl.program_id(0)
    f = pl.program_id(1)
    a = y_ref[0, 0] * we_ref[...]
    m = jnp.dot(x_ref[b, :, f, :], a,
                preferred_element_type=jnp.float32)
    m = m * jnp.float32(1.0 / _N)
    h = jnp.dot(wl_ref[...], m,
                preferred_element_type=jnp.float32) + b_ref[...]
    o_ref[b, :, f, :] = jnp.maximum(h, 0.0)


@jax.jit
def kernel(Y, infos, W_edge, W_lin, b_lin):
    b2 = b_lin.reshape(_C, 1)
    grid = (_B, _F)
    return pl.pallas_call(
        _body,
        grid=grid,
        in_specs=[
            pl.BlockSpec((1, 1, _N, _N), lambda b, f: (b, f, 0, 0)),
            pl.BlockSpec((_B, _C, _F, _N), lambda b, f: (0, 0, 0, 0)),
            pl.BlockSpec((_N, _N), lambda b, f: (0, 0)),
            pl.BlockSpec((_C, _C), lambda b, f: (0, 0)),
            pl.BlockSpec((_C, 1), lambda b, f: (0, 0)),
        ],
        out_specs=pl.BlockSpec((_B, _C, _F, _N), lambda b, f: (0, 0, 0, 0)),
        out_shape=jax.ShapeDtypeStruct((_B, _C, _F, _N), jnp.float32),
    )(Y, infos, W_edge, W_lin, b2)
